# Initial kernel scaffold; baseline (speedup 1.0000x reference)
import jax
import jax.numpy as jnp
from jax.experimental import pallas as pl


def kernel(user_n_id, book_n_id, book_x, edge_index, edge_label_index, lin_W, lin_b, user_emb, book_emb, c1_rates_Wl, c1_rates_bl, c1_rates_Wr, c1_rev_Wl, c1_rev_bl, c1_rev_Wr, c2_rates_Wl, c2_rates_bl, c2_rates_Wr, c2_rev_Wl, c2_rev_bl, c2_rev_Wr):
    raise NotImplementedError("write your pallas kernel here")


# speedup vs baseline: 4.1384x; 4.1384x over previous
import functools

import jax
import jax.numpy as jnp
from jax import lax
from jax.experimental import pallas as pl
from jax.experimental.pallas import tpu as pltpu
from jax.experimental.pallas import tpu_sc as plsc

N = 10000
NPAD = 10240
HID = 256
H = 128
E = 160000
EPAD = 163840
NC, NS = 2, 16
NW = NC * NS
K = 128

@functools.lru_cache(maxsize=None)
def _mesh():
    return plsc.VectorSubcoreMesh(
        core_axis_name="c", subcore_axis_name="s",
        num_cores=NC, num_subcores=NS)


def _fill(ref, nrows, ncols, val):
    v = jnp.full((16,), val, jnp.float32)
    per_row = ncols // 16

    def body(i, _):
        r = i // per_row
        j = (i % per_row) * 16
        ref[r, pl.ds(j, 16)] = v
        return 0

    lax.fori_loop(0, nrows * per_row, body, 0)


def _seg_body(x_hbm, src_hbm, dst_hbm, out_hbm, src_v, dst_v, rows_v, acc_sh,
              sem):
    c = lax.axis_index("c")
    s = lax.axis_index("s")
    chunks_per_tile = EPAD // K // NS
    rows_per_tile = NPAD // NS

    pltpu.sync_copy(src_hbm.at[pl.ds(s * chunks_per_tile, chunks_per_tile)],
                    src_v)
    pltpu.sync_copy(dst_hbm.at[pl.ds(s * chunks_per_tile, chunks_per_tile)],
                    dst_v)

    _fill(rows_v, K, H, 0.0)
    for t in range(rows_per_tile // K):
        pltpu.sync_copy(rows_v, acc_sh.at[pl.ds(s * rows_per_tile + t * K, K)])
    plsc.subcore_barrier()

    def chunk(g, _):
        pltpu.async_copy(x_hbm.at[c].at[src_v.at[g]], rows_v, sem).wait()
        pltpu.sync_copy(rows_v, acc_sh.at[dst_v.at[g]], add=True)
        return 0

    lax.fori_loop(0, chunks_per_tile, chunk, 0)
    plsc.subcore_barrier()

    pltpu.sync_copy(acc_sh.at[pl.ds(s * rows_per_tile, rows_per_tile)],
                    out_hbm.at[c].at[pl.ds(s * rows_per_tile, rows_per_tile)])


@functools.lru_cache(maxsize=None)
def _seg_k():
  return pl.kernel(
    _seg_body,
    out_type=jax.ShapeDtypeStruct((NC, NPAD, H), jnp.float32),
    mesh=_mesh(),
    compiler_params=pltpu.CompilerParams(needs_layout_passes=False),
    scratch_types=[
        pltpu.VMEM((EPAD // K // NS, K), jnp.int32),
        pltpu.VMEM((EPAD // K // NS, K), jnp.int32),
        pltpu.VMEM((K, H), jnp.float32),
        pltpu.VMEM_SHARED((NPAD, H), jnp.float32),
        pltpu.SemaphoreType.DMA,
    ])


def _cnt_body(idx2_hbm, out_hbm, idx_v, ones_v, acc_sh):
    c = lax.axis_index("c")
    s = lax.axis_index("s")
    chunks_per_tile = EPAD // K // NS
    rows_per_tile = NPAD // NS

    pltpu.sync_copy(
        idx2_hbm.at[c].at[pl.ds(s * chunks_per_tile, chunks_per_tile)], idx_v)

    _fill(ones_v, K, 16, 0.0)
    for t in range(rows_per_tile // K):
        pltpu.sync_copy(ones_v, acc_sh.at[pl.ds(s * rows_per_tile + t * K, K)])
    _fill(ones_v, K, 16, 1.0)
    plsc.subcore_barrier()

    def chunk(g, _):
        pltpu.sync_copy(ones_v, acc_sh.at[idx_v.at[g]], add=True)
        return 0

    lax.fori_loop(0, chunks_per_tile, chunk, 0)
    plsc.subcore_barrier()

    pltpu.sync_copy(acc_sh.at[pl.ds(s * rows_per_tile, rows_per_tile)],
                    out_hbm.at[c].at[pl.ds(s * rows_per_tile, rows_per_tile)])


@functools.lru_cache(maxsize=None)
def _cnt_k():
  return pl.kernel(
    _cnt_body,
    out_type=jax.ShapeDtypeStruct((NC, NPAD, 16), jnp.float32),
    mesh=_mesh(),
    compiler_params=pltpu.CompilerParams(
        needs_layout_passes=False, use_tc_tiling_on_sc=False),
    scratch_types=[
        pltpu.VMEM((EPAD // K // NS, K), jnp.int32),
        pltpu.VMEM((K, 16), jnp.float32),
        pltpu.VMEM_SHARED((NPAD, 16), jnp.float32),
    ])


_GK = 80


def _gat_body(tbl_hbm, idx_hbm, out_hbm, idx_v, rows_v, sem):
    c = lax.axis_index("c")
    s = lax.axis_index("s")
    w = s * NC + c
    chunks_per_tile = NPAD // _GK // NW

    pltpu.sync_copy(idx_hbm.at[pl.ds(w * chunks_per_tile, chunks_per_tile)],
                    idx_v)
    for t in range(chunks_per_tile):
        pltpu.async_copy(tbl_hbm.at[idx_v.at[t]], rows_v, sem).wait()
        base = w * chunks_per_tile * _GK + t * _GK
        pltpu.sync_copy(rows_v.at[pl.ds(0, _GK), pl.ds(0, H)],
                        out_hbm.at[0].at[pl.ds(base, _GK)])
        pltpu.sync_copy(rows_v.at[pl.ds(0, _GK), pl.ds(H, H)],
                        out_hbm.at[1].at[pl.ds(base, _GK)])


@functools.lru_cache(maxsize=None)
def _gat_k():
  return pl.kernel(
    _gat_body,
    out_type=jax.ShapeDtypeStruct((NC, NPAD, H), jnp.float32),
    mesh=_mesh(),
    compiler_params=pltpu.CompilerParams(needs_layout_passes=False),
    scratch_types=[
        pltpu.VMEM((NPAD // _GK // NW, _GK), jnp.int32),
        pltpu.VMEM((_GK, 2 * H), jnp.float32),
        pltpu.SemaphoreType.DMA,
    ])


def _dec_body(zu_hbm, zb_hbm, row_hbm, col_hbm, out_hbm, ri_v, ci_v, a0, a1,
              b0, b1, pacc, ov, sem):
    c = lax.axis_index("c")
    s = lax.axis_index("s")
    w = s * NC + c
    chunks_per_tile = EPAD // K // NW

    pltpu.sync_copy(row_hbm.at[pl.ds(w * chunks_per_tile, chunks_per_tile)],
                    ri_v)
    pltpu.sync_copy(col_hbm.at[pl.ds(w * chunks_per_tile, chunks_per_tile)],
                    ci_v)

    def chunk(g, _):
        d0 = pltpu.async_copy(zu_hbm.at[0].at[ri_v.at[g]], a0, sem)
        d1 = pltpu.async_copy(zu_hbm.at[1].at[ri_v.at[g]], a1, sem)
        d2 = pltpu.async_copy(zb_hbm.at[0].at[ci_v.at[g]], b0, sem)
        d3 = pltpu.async_copy(zb_hbm.at[1].at[ci_v.at[g]], b1, sem)
        d0.wait(); d1.wait(); d2.wait(); d3.wait()

        def ebody(e, _):
            acc = jnp.zeros((16,), jnp.float32)
            for k in range(H // 16):
                sl = pl.ds(k * 16, 16)
                acc = acc + a0[e, sl] * b0[e, sl]
                acc = acc + a1[e, sl] * b1[e, sl]
            pacc[pl.ds(e * 16, 16)] = acc
            return 0

        lax.fori_loop(0, K, ebody, 0)

        def gbody(grp, _):
            e_v = (lax.iota(jnp.int32, 16) + grp * 16) * 16
            out_v = jnp.zeros((16,), jnp.float32)
            for k in range(16):
                out_v = out_v + plsc.load_gather(pacc, [e_v + k])
            ov[pl.ds(grp * 16, 16)] = out_v
            return 0

        lax.fori_loop(0, K // 16, gbody, 0)
        pltpu.sync_copy(
            ov, out_hbm.at[pl.ds(w * chunks_per_tile * K + g * K, K)])
        return 0

    lax.fori_loop(0, chunks_per_tile, chunk, 0)


@functools.lru_cache(maxsize=None)
def _dec_k():
  return pl.kernel(
    _dec_body,
    out_type=jax.ShapeDtypeStruct((EPAD,), jnp.float32),
    mesh=_mesh(),
    compiler_params=pltpu.CompilerParams(needs_layout_passes=False),
    scratch_types=[
        pltpu.VMEM((EPAD // K // NW, K), jnp.int32),
        pltpu.VMEM((EPAD // K // NW, K), jnp.int32),
        pltpu.VMEM((K, H), jnp.float32),
        pltpu.VMEM((K, H), jnp.float32),
        pltpu.VMEM((K, H), jnp.float32),
        pltpu.VMEM((K, H), jnp.float32),
        pltpu.VMEM((K * 16,), jnp.float32),
        pltpu.VMEM((K,), jnp.float32),
        pltpu.SemaphoreType.DMA,
    ])


_R = 1024


def _enc_body(bx_ref, w_ref, b_ref, be_ref, o_ref):
    acc = jnp.dot(bx_ref[...], w_ref[...], preferred_element_type=jnp.float32)
    o_ref[0] = acc + b_ref[0, 0][None, :] + be_ref[0]


def _enc_tc(bx, lin_W, lin_b2, be):
    return pl.pallas_call(
        _enc_body,
        grid=(NPAD // _R, NC),
        in_specs=[
            pl.BlockSpec((_R, 384), lambda r, c: (r, 0)),
            pl.BlockSpec((384, H), lambda r, c: (0, c)),
            pl.BlockSpec((1, 1, H), lambda r, c: (c, 0, 0)),
            pl.BlockSpec((1, _R, H), lambda r, c: (c, r, 0)),
        ],
        out_specs=pl.BlockSpec((1, _R, H), lambda r, c: (c, r, 0)),
        out_shape=jax.ShapeDtypeStruct((NC, NPAD, H), jnp.float32),
    )(bx, lin_W, lin_b2, be)


def _comb_body(agg_ref, x_ref, wl_ref, wr_ref, bl_ref, cnt_ref, o_ref, *,
               relu):
    r = 1.0 / jnp.maximum(cnt_ref[:, 0:1], 1.0)
    acc = jnp.dot(agg_ref[0] * r, wl_ref[:H], preferred_element_type=jnp.float32)
    acc = acc + jnp.dot(agg_ref[1] * r, wl_ref[H:],
                        preferred_element_type=jnp.float32)
    acc = acc + jnp.dot(x_ref[0], wr_ref[:H], preferred_element_type=jnp.float32)
    acc = acc + jnp.dot(x_ref[1], wr_ref[H:], preferred_element_type=jnp.float32)
    acc = acc + bl_ref[0, 0][None, :]
    if relu:
        acc = jnp.maximum(acc, 0.0)
    o_ref[0] = acc


def _comb_tc(agg, x, Wl, Wr, bl2, cnt, relu):
    return pl.pallas_call(
        functools.partial(_comb_body, relu=relu),
        grid=(NPAD // _R, NC),
        in_specs=[
            pl.BlockSpec((NC, _R, H), lambda r, c: (0, r, 0)),
            pl.BlockSpec((NC, _R, H), lambda r, c: (0, r, 0)),
            pl.BlockSpec((HID, H), lambda r, c: (0, c)),
            pl.BlockSpec((HID, H), lambda r, c: (0, c)),
            pl.BlockSpec((1, 1, H), lambda r, c: (c, 0, 0)),
            pl.BlockSpec((_R, 16), lambda r, c: (r, 0)),
        ],
        out_specs=pl.BlockSpec((1, _R, H), lambda r, c: (c, r, 0)),
        out_shape=jax.ShapeDtypeStruct((NC, NPAD, H), jnp.float32),
    )(agg, x, Wl, Wr, bl2, cnt)


def _pad_idx(idx, n_to, lo, span):
    npad = n_to - idx.shape[0]
    pad = lo + (jnp.arange(npad, dtype=jnp.int32) % span)
    return jnp.concatenate([idx.astype(jnp.int32), pad])


def kernel(user_n_id, book_n_id, book_x, edge_index, edge_label_index,
           lin_W, lin_b, user_emb, book_emb,
           c1_rates_Wl, c1_rates_bl, c1_rates_Wr,
           c1_rev_Wl, c1_rev_bl, c1_rev_Wr,
           c2_rates_Wl, c2_rates_bl, c2_rates_Wr,
           c2_rev_Wl, c2_rev_bl, c2_rev_Wr):
    uid = _pad_idx(user_n_id, NPAD, 0, 240).reshape(NPAD // _GK, _GK)
    bid = _pad_idx(book_n_id, NPAD, 0, 240).reshape(NPAD // _GK, _GK)
    src = _pad_idx(edge_index[0], EPAD, N, 240).reshape(EPAD // K, K)
    dst = _pad_idx(edge_index[1], EPAD, N, 240).reshape(EPAD // K, K)
    row = _pad_idx(edge_label_index[0], EPAD, N, 240).reshape(EPAD // K, K)
    col = _pad_idx(edge_label_index[1], EPAD, N, 240).reshape(EPAD // K, K)
    idx2 = jnp.stack([dst, src])

    bx = jnp.pad(book_x, ((0, NPAD - N), (0, 0)))
    lin_b2 = lin_b.reshape(NC, 1, H)

    xu = _gat_k()(user_emb, uid)
    be = _gat_k()(book_emb, bid)
    xb = _enc_tc(bx, lin_W, lin_b2, be)
    cnt = _cnt_k()(idx2)
    cnt_d, cnt_s = cnt[0], cnt[1]

    aggb = _seg_k()(xu, src, dst)
    aggu = _seg_k()(xb, dst, src)
    hb = _comb_tc(aggb, xb, c1_rates_Wl, c1_rates_Wr,
                  c1_rates_bl.reshape(NC, 1, H), cnt_d, True)
    hu = _comb_tc(aggu, xu, c1_rev_Wl, c1_rev_Wr,
                  c1_rev_bl.reshape(NC, 1, H), cnt_s, True)

    aggb2 = _seg_k()(hu, src, dst)
    aggu2 = _seg_k()(hb, dst, src)
    zb = _comb_tc(aggb2, hb, c2_rates_Wl, c2_rates_Wr,
                  c2_rates_bl.reshape(NC, 1, H), cnt_d, False)
    zu = _comb_tc(aggu2, hu, c2_rev_Wl, c2_rev_Wr,
                  c2_rev_bl.reshape(NC, 1, H), cnt_s, False)

    return _dec_k()(zu, zb, row, col)[:E]
